# baseline (speedup 1.0000x reference)
import functools

import jax
import jax.numpy as jnp
from jax import lax
from jax.experimental import pallas as pl
from jax.experimental.pallas import tpu as pltpu
from jax.experimental.pallas import tpu_sc as plsc

N_NODES = 10000
D = 128
ROWS_PAD = 10240
PAD_IDX = N_NODES
NC, NS, L = 2, 16, 16
NW = NC * NS
CHUNK = 128
ROWS_PER_TILE = ROWS_PAD // NS

SLAB = 128
SLABP = 128
E_ADJ = 320000
DS = -(-E_ADJ // (NW * SLAB))
MS = 2 * (-(-E_ADJ // (NS * SLAB * 2)))
E_PAIR = 200000
PS = -(-E_PAIR // (NW * SLABP))
DH = D // 2


@functools.cache
def _sc_kernels():
    mesh = plsc.VectorSubcoreMesh(core_axis_name="c", subcore_axis_name="s")

    params = pltpu.CompilerParams(use_tc_tiling_on_sc=False)

    deg = functools.partial(
        pl.kernel, mesh=mesh, compiler_params=params,
        out_type=jax.ShapeDtypeStruct((NC, ROWS_PAD, 16), jnp.float32),
        scratch_types=[
            pltpu.VMEM((DS, SLAB), jnp.int32),
            pltpu.VMEM((SLAB, 16), jnp.float32),
            pltpu.VMEM((ROWS_PER_TILE, 16), jnp.float32),
            pltpu.VMEM_SHARED((ROWS_PAD, 16), jnp.float32),
        ])(_deg_body)

    msg = functools.partial(
        pl.kernel, mesh=mesh, compiler_params=params,
        out_type=jax.ShapeDtypeStruct((NC, ROWS_PAD, DH), jnp.float32),
        scratch_types=[
            pltpu.VMEM((MS, SLAB), jnp.int32),
            pltpu.VMEM((MS, SLAB), jnp.int32),
            pltpu.VMEM((SLAB, DH), jnp.float32),
            pltpu.VMEM((CHUNK, DH), jnp.float32),
            pltpu.VMEM_SHARED((ROWS_PAD, DH), jnp.float32),
            pltpu.SemaphoreType.DMA,
        ])(_msg_body)

    pair = functools.partial(
        pl.kernel, mesh=mesh, compiler_params=params,
        out_type=[jax.ShapeDtypeStruct((NW * PS * SLABP, D), jnp.float32),
                  jax.ShapeDtypeStruct((NW * PS * SLABP, D), jnp.float32)],
        scratch_types=[
            pltpu.VMEM((PS, SLABP), jnp.int32),
            pltpu.VMEM((PS, SLABP), jnp.int32),
            pltpu.VMEM((SLABP, D), jnp.float32),
            pltpu.VMEM((SLABP, D), jnp.float32),
            pltpu.SemaphoreType.DMA,
            pltpu.SemaphoreType.DMA,
        ])(_pair_body)

    return deg, msg, pair


def _deg_body(dst_hbm, out_hbm, dst_v, ones_v, z_v, acc):
    c = lax.axis_index("c")
    s = lax.axis_index("s")
    w = s * NC + c

    def fill_ones(i, carry):
        ones_v[i, :] = jnp.ones((L,), jnp.float32)
        return carry
    lax.fori_loop(0, SLAB, fill_ones, 0)

    def fill_zero(i, carry):
        z_v[i, :] = jnp.zeros((L,), jnp.float32)
        return carry
    lax.fori_loop(0, ROWS_PER_TILE, fill_zero, 0)

    pltpu.sync_copy(z_v, acc.at[pl.ds(s * ROWS_PER_TILE, ROWS_PER_TILE)])
    pltpu.sync_copy(dst_hbm.at[w], dst_v)
    plsc.subcore_barrier()

    def body(j, carry):
        pltpu.sync_copy(ones_v, acc.at[dst_v.at[j]], add=True)
        return carry
    lax.fori_loop(0, DS, body, 0)

    plsc.subcore_barrier()
    sl = pl.ds(s * ROWS_PER_TILE, ROWS_PER_TILE)
    pltpu.sync_copy(acc.at[sl], out_hbm.at[c, sl])


def _msg_body(g2_hbm, src2_hbm, dst_hbm, out_hbm, src_v, dst_v, rows_v, z_v,
              acc, sem):
    c = lax.axis_index("c")
    s = lax.axis_index("s")

    def fill_zero(i, carry):
        for col in range(DH // L):
            z_v[i, pl.ds(col * L, L)] = jnp.zeros((L,), jnp.float32)
        return carry
    lax.fori_loop(0, CHUNK, fill_zero, 0)

    for r in range(ROWS_PER_TILE // CHUNK):
        pltpu.sync_copy(z_v, acc.at[pl.ds(s * ROWS_PER_TILE + r * CHUNK, CHUNK)])
    pltpu.sync_copy(src2_hbm.at[c, s], src_v)
    pltpu.sync_copy(dst_hbm.at[s], dst_v)
    plsc.subcore_barrier()

    def body(j, carry):
        pltpu.async_copy(g2_hbm.at[src_v.at[j]], rows_v, sem).wait()
        pltpu.sync_copy(rows_v, acc.at[dst_v.at[j]], add=True)
        return carry
    lax.fori_loop(0, MS, body, 0)

    plsc.subcore_barrier()
    sl = pl.ds(s * ROWS_PER_TILE, ROWS_PER_TILE)
    pltpu.sync_copy(acc.at[sl], out_hbm.at[c, sl])


def _pair_body(p_hbm, q_hbm, a_hbm, b_hbm, za_hbm, zb_hbm,
               a_v, b_v, bufa, bufb, sga, sgb):
    c = lax.axis_index("c")
    s = lax.axis_index("s")
    w = s * NC + c
    base = w * PS * SLABP

    pltpu.sync_copy(a_hbm.at[w], a_v)
    pltpu.sync_copy(b_hbm.at[w], b_v)

    def body(j, carry):
        cpa = pltpu.async_copy(p_hbm.at[a_v.at[j]], bufa, sga)
        cpb = pltpu.async_copy(q_hbm.at[b_v.at[j]], bufb, sgb)
        sl = pl.ds(base + j * SLABP, SLABP)
        cpa.wait()
        pltpu.sync_copy(bufa, za_hbm.at[sl])
        cpb.wait()
        pltpu.sync_copy(bufb, zb_hbm.at[sl])
        return carry
    lax.fori_loop(0, PS, body, 0)


_BLK = 1024


def _first_layer_body(degp_ref, x_ref, w_ref, dinv_ref, g_ref):
    i = pl.program_id(0)
    deg = degp_ref[0, :, 0:1] + degp_ref[1, :, 0:1] + 1.0
    row = i * _BLK + lax.broadcasted_iota(jnp.int32, (_BLK, 1), 0)
    dinv = jnp.where(row < N_NODES, lax.rsqrt(deg), 0.0)
    dinv_b = jnp.broadcast_to(dinv, (_BLK, D))
    dinv_ref[...] = dinv_b
    g_ref[...] = dinv_b * jnp.dot(x_ref[...], w_ref[...],
                                  preferred_element_type=jnp.float32)


def _first_layer(degp, x_pad, w0):
    return pl.pallas_call(
        _first_layer_body,
        grid=(ROWS_PAD // _BLK,),
        in_specs=[
            pl.BlockSpec((NC, _BLK, 16), lambda i: (0, i, 0)),
            pl.BlockSpec((_BLK, D), lambda i: (i, 0)),
            pl.BlockSpec((D, D), lambda i: (0, 0)),
        ],
        out_specs=[
            pl.BlockSpec((_BLK, D), lambda i: (i, 0)),
            pl.BlockSpec((_BLK, D), lambda i: (i, 0)),
        ],
        out_shape=[
            jax.ShapeDtypeStruct((ROWS_PAD, D), jnp.float32),
            jax.ShapeDtypeStruct((ROWS_PAD, D), jnp.float32),
        ],
    )(degp, x_pad, w0)


def _mid_layer_body(g_ref, p_ref, b_ref, dinv_ref, w_ref, out_ref):
    agg = g_ref[...] + jnp.concatenate([p_ref[0], p_ref[1]], axis=-1)
    h = jnp.maximum(dinv_ref[...] * agg + b_ref[...], 0.0)
    out_ref[...] = dinv_ref[...] * jnp.dot(h, w_ref[...],
                                           preferred_element_type=jnp.float32)


def _mid_layer(g, part, bias, dinv_b, w_next):
    return pl.pallas_call(
        _mid_layer_body,
        grid=(ROWS_PAD // _BLK,),
        in_specs=[
            pl.BlockSpec((_BLK, D), lambda i: (i, 0)),
            pl.BlockSpec((NC, _BLK, DH), lambda i: (0, i, 0)),
            pl.BlockSpec((1, D), lambda i: (0, 0)),
            pl.BlockSpec((_BLK, D), lambda i: (i, 0)),
            pl.BlockSpec((D, D), lambda i: (0, 0)),
        ],
        out_specs=pl.BlockSpec((_BLK, D), lambda i: (i, 0)),
        out_shape=jax.ShapeDtypeStruct((ROWS_PAD, D), jnp.float32),
    )(g, part, bias.reshape(1, D), dinv_b, w_next)


def _last_layer_body(g_ref, p_ref, b_ref, dinv_ref, wa_ref, wb_ref, bl_ref,
                     p_out, q_out):
    agg = g_ref[...] + jnp.concatenate([p_ref[0], p_ref[1]], axis=-1)
    h = dinv_ref[...] * agg + b_ref[...]
    p_out[...] = jnp.dot(h, wa_ref[...], preferred_element_type=jnp.float32)
    q_out[...] = jnp.dot(h, wb_ref[...],
                         preferred_element_type=jnp.float32) + bl_ref[...]


def _last_layer(g, part, bias, dinv_b, wa, wb, bl1):
    return pl.pallas_call(
        _last_layer_body,
        grid=(ROWS_PAD // _BLK,),
        in_specs=[
            pl.BlockSpec((_BLK, D), lambda i: (i, 0)),
            pl.BlockSpec((NC, _BLK, DH), lambda i: (0, i, 0)),
            pl.BlockSpec((1, D), lambda i: (0, 0)),
            pl.BlockSpec((_BLK, D), lambda i: (i, 0)),
            pl.BlockSpec((D, D), lambda i: (0, 0)),
            pl.BlockSpec((D, D), lambda i: (0, 0)),
            pl.BlockSpec((1, D), lambda i: (0, 0)),
        ],
        out_specs=[
            pl.BlockSpec((_BLK, D), lambda i: (i, 0)),
            pl.BlockSpec((_BLK, D), lambda i: (i, 0)),
        ],
        out_shape=[
            jax.ShapeDtypeStruct((ROWS_PAD, D), jnp.float32),
            jax.ShapeDtypeStruct((ROWS_PAD, D), jnp.float32),
        ],
    )(g, part, bias.reshape(1, D), dinv_b, wa, wb, bl1.reshape(1, D))


def _head_body(za_ref, zb_ref, w_ref, b_ref, out_ref):
    u = jnp.maximum(za_ref[...] + zb_ref[...], 0.0)
    logits = jnp.dot(u, w_ref[...],
                     preferred_element_type=jnp.float32) + b_ref[...]
    m = jnp.max(logits, axis=-1, keepdims=True)
    lse = m + jnp.log(jnp.sum(jnp.exp(logits - m), axis=-1, keepdims=True))
    out_ref[...] = logits - lse


def _head(za, zb, wl2, bl2):
    f_pad = za.shape[0]
    return pl.pallas_call(
        _head_body,
        grid=(f_pad // _BLK,),
        in_specs=[
            pl.BlockSpec((_BLK, D), lambda i: (i, 0)),
            pl.BlockSpec((_BLK, D), lambda i: (i, 0)),
            pl.BlockSpec((D, 2), lambda i: (0, 0)),
            pl.BlockSpec((1, 2), lambda i: (0, 0)),
        ],
        out_specs=pl.BlockSpec((_BLK, 2), lambda i: (i, 0)),
        out_shape=jax.ShapeDtypeStruct((f_pad, 2), jnp.float32),
    )(za, zb, wl2, bl2.reshape(1, 2))


def _prep_idx(idx, n_workers, n_slabs, slab):
    total = n_workers * n_slabs * slab
    idx = idx.astype(jnp.int32)
    idx = jnp.pad(idx, (0, total - idx.shape[0]), constant_values=PAD_IDX)
    return idx.reshape(n_workers, n_slabs, slab)


def kernel(x, edge_index, pos_edges_train, neg_edges_train, pos_edges_test,
           neg_edges_test, W0, b0, W1, b1, W2, b2, W3, b3, Wl1, bl1, Wl2, bl2):
    dst32 = _prep_idx(edge_index[1], NW, DS, SLAB)
    src_m = _prep_idx(edge_index[0], NS, MS, SLAB)
    src2 = jnp.stack([2 * src_m, 2 * src_m + 1])
    dst_m = _prep_idx(edge_index[1], NS, MS, SLAB)
    a_idx = _prep_idx(jnp.concatenate([
        pos_edges_train[0], neg_edges_train[0],
        pos_edges_test[0], neg_edges_test[0]]), NW, PS, SLABP)
    b_idx = _prep_idx(jnp.concatenate([
        pos_edges_train[1], neg_edges_train[1],
        pos_edges_test[1], neg_edges_test[1]]), NW, PS, SLABP)

    x_pad = jnp.pad(x, ((0, ROWS_PAD - N_NODES), (0, 0)))

    deg_k, msg_k, pair_k = _sc_kernels()

    degp = deg_k(dst32)
    dinv_b, g = _first_layer(degp, x_pad, W0)

    def g2(arr):
        return arr.reshape(2 * ROWS_PAD, DH)

    part = msg_k(g2(g), src2, dst_m)
    g = _mid_layer(g, part, b0, dinv_b, W1)
    part = msg_k(g2(g), src2, dst_m)
    g = _mid_layer(g, part, b1, dinv_b, W2)
    part = msg_k(g2(g), src2, dst_m)
    g = _mid_layer(g, part, b2, dinv_b, W3)
    part = msg_k(g2(g), src2, dst_m)
    p_tab, q_tab = _last_layer(g, part, b3, dinv_b, Wl1[:D], Wl1[D:], bl1)

    za, zb = pair_k(p_tab, q_tab, a_idx, b_idx)
    z = _head(za, zb, Wl2, bl2)

    n_train = 2 * pos_edges_train.shape[1]
    n_total = n_train + 2 * pos_edges_test.shape[1]
    return (z[:n_train], z[n_train:n_total])


# speedup vs baseline: 1.1510x; 1.1510x over previous
import functools

import jax
import jax.numpy as jnp
from jax import lax
from jax.experimental import pallas as pl
from jax.experimental.pallas import tpu as pltpu
from jax.experimental.pallas import tpu_sc as plsc

N_NODES = 10000
D = 128
ROWS_PAD = 10240
PAD_IDX = N_NODES
NC, NS, L = 2, 16, 16
NW = NC * NS
CHUNK = 128
ROWS_PER_TILE = ROWS_PAD // NS

SLAB = 128
SLABP = 128
E_ADJ = 320000
DS = -(-E_ADJ // (NW * SLAB))
MS = -(-E_ADJ // (NS * SLAB))
E_PAIR = 200000
PS = -(-E_PAIR // (NW * SLABP))
DH = D // 2


@functools.cache
def _sc_kernels():
    mesh = plsc.VectorSubcoreMesh(core_axis_name="c", subcore_axis_name="s")

    params = pltpu.CompilerParams(use_tc_tiling_on_sc=False)

    deg = functools.partial(
        pl.kernel, mesh=mesh, compiler_params=params,
        out_type=jax.ShapeDtypeStruct((NC, ROWS_PAD, 16), jnp.float32),
        scratch_types=[
            pltpu.VMEM((DS, SLAB), jnp.int32),
            pltpu.VMEM((SLAB, 16), jnp.float32),
            pltpu.VMEM((ROWS_PER_TILE, 16), jnp.float32),
            pltpu.VMEM_SHARED((ROWS_PAD, 16), jnp.float32),
        ])(_deg_body)

    msg = functools.partial(
        pl.kernel, mesh=mesh, compiler_params=params,
        out_type=jax.ShapeDtypeStruct((NC, ROWS_PAD, DH), jnp.float32),
        scratch_types=[
            pltpu.VMEM((MS, SLAB), jnp.int32),
            pltpu.VMEM((MS, SLAB), jnp.int32),
            pltpu.VMEM((SLAB, DH), jnp.float32),
            pltpu.VMEM((CHUNK, DH), jnp.float32),
            pltpu.VMEM_SHARED((ROWS_PAD, DH), jnp.float32),
            pltpu.SemaphoreType.DMA,
        ])(_msg_body)

    pair = functools.partial(
        pl.kernel, mesh=mesh, compiler_params=params,
        out_type=[jax.ShapeDtypeStruct((NW * PS * SLABP, D), jnp.float32),
                  jax.ShapeDtypeStruct((NW * PS * SLABP, D), jnp.float32)],
        scratch_types=[
            pltpu.VMEM((PS, SLABP), jnp.int32),
            pltpu.VMEM((PS, SLABP), jnp.int32),
            pltpu.VMEM((SLABP, D), jnp.float32),
            pltpu.VMEM((SLABP, D), jnp.float32),
            pltpu.SemaphoreType.DMA,
            pltpu.SemaphoreType.DMA,
        ])(_pair_body)

    return deg, msg, pair


def _deg_body(dst_hbm, out_hbm, dst_v, ones_v, z_v, acc):
    c = lax.axis_index("c")
    s = lax.axis_index("s")
    w = s * NC + c

    def fill_ones(i, carry):
        ones_v[i, :] = jnp.ones((L,), jnp.float32)
        return carry
    lax.fori_loop(0, SLAB, fill_ones, 0)

    def fill_zero(i, carry):
        z_v[i, :] = jnp.zeros((L,), jnp.float32)
        return carry
    lax.fori_loop(0, ROWS_PER_TILE, fill_zero, 0)

    pltpu.sync_copy(z_v, acc.at[pl.ds(s * ROWS_PER_TILE, ROWS_PER_TILE)])
    pltpu.sync_copy(dst_hbm.at[w], dst_v)
    plsc.subcore_barrier()

    def body(j, carry):
        pltpu.sync_copy(ones_v, acc.at[dst_v.at[j]], add=True)
        return carry
    lax.fori_loop(0, DS, body, 0)

    plsc.subcore_barrier()
    sl = pl.ds(s * ROWS_PER_TILE, ROWS_PER_TILE)
    pltpu.sync_copy(acc.at[sl], out_hbm.at[c, sl])


def _msg_body(g2_hbm, src2_hbm, dst_hbm, out_hbm, src_v, dst_v, rows_v, z_v,
              acc, sem):
    c = lax.axis_index("c")
    s = lax.axis_index("s")

    def fill_zero(i, carry):
        for col in range(DH // L):
            z_v[i, pl.ds(col * L, L)] = jnp.zeros((L,), jnp.float32)
        return carry
    lax.fori_loop(0, CHUNK, fill_zero, 0)

    for r in range(ROWS_PER_TILE // CHUNK):
        pltpu.sync_copy(z_v, acc.at[pl.ds(s * ROWS_PER_TILE + r * CHUNK, CHUNK)])
    pltpu.sync_copy(src2_hbm.at[c, s], src_v)
    pltpu.sync_copy(dst_hbm.at[s], dst_v)
    plsc.subcore_barrier()

    def body(j, carry):
        pltpu.async_copy(g2_hbm.at[src_v.at[j]], rows_v, sem).wait()
        pltpu.sync_copy(rows_v, acc.at[dst_v.at[j]], add=True)
        return carry
    lax.fori_loop(0, MS, body, 0)

    plsc.subcore_barrier()
    sl = pl.ds(s * ROWS_PER_TILE, ROWS_PER_TILE)
    pltpu.sync_copy(acc.at[sl], out_hbm.at[c, sl])


def _pair_body(p_hbm, q_hbm, a_hbm, b_hbm, za_hbm, zb_hbm,
               a_v, b_v, bufa, bufb, sga, sgb):
    c = lax.axis_index("c")
    s = lax.axis_index("s")
    w = s * NC + c
    base = w * PS * SLABP

    pltpu.sync_copy(a_hbm.at[w], a_v)
    pltpu.sync_copy(b_hbm.at[w], b_v)

    def body(j, carry):
        cpa = pltpu.async_copy(p_hbm.at[a_v.at[j]], bufa, sga)
        cpb = pltpu.async_copy(q_hbm.at[b_v.at[j]], bufb, sgb)
        sl = pl.ds(base + j * SLABP, SLABP)
        cpa.wait()
        cpb.wait()
        pltpu.sync_copy(bufa, za_hbm.at[sl])
        pltpu.sync_copy(bufb, zb_hbm.at[sl])
        return carry
    lax.fori_loop(0, PS, body, 0)


_BLK = 1024


def _first_layer_body(degp_ref, x_ref, w_ref, dinv_ref, g_ref):
    i = pl.program_id(0)
    deg = degp_ref[0, :, 0:1] + degp_ref[1, :, 0:1] + 1.0
    row = i * _BLK + lax.broadcasted_iota(jnp.int32, (_BLK, 1), 0)
    dinv = jnp.where(row < N_NODES, lax.rsqrt(deg), 0.0)
    dinv_b = jnp.broadcast_to(dinv, (_BLK, D))
    dinv_ref[...] = dinv_b
    g_ref[...] = dinv_b * jnp.dot(x_ref[...], w_ref[...],
                                  preferred_element_type=jnp.float32)


def _first_layer(degp, x_pad, w0):
    return pl.pallas_call(
        _first_layer_body,
        grid=(ROWS_PAD // _BLK,),
        in_specs=[
            pl.BlockSpec((NC, _BLK, 16), lambda i: (0, i, 0)),
            pl.BlockSpec((_BLK, D), lambda i: (i, 0)),
            pl.BlockSpec((D, D), lambda i: (0, 0)),
        ],
        out_specs=[
            pl.BlockSpec((_BLK, D), lambda i: (i, 0)),
            pl.BlockSpec((_BLK, D), lambda i: (i, 0)),
        ],
        out_shape=[
            jax.ShapeDtypeStruct((ROWS_PAD, D), jnp.float32),
            jax.ShapeDtypeStruct((ROWS_PAD, D), jnp.float32),
        ],
    )(degp, x_pad, w0)


def _mid_layer_body(g_ref, p_ref, b_ref, dinv_ref, w_ref, out_ref):
    agg = g_ref[...] + jnp.concatenate([p_ref[0], p_ref[1]], axis=-1)
    h = jnp.maximum(dinv_ref[...] * agg + b_ref[...], 0.0)
    out_ref[...] = dinv_ref[...] * jnp.dot(h, w_ref[...],
                                           preferred_element_type=jnp.float32)


def _mid_layer(g, part, bias, dinv_b, w_next):
    return pl.pallas_call(
        _mid_layer_body,
        grid=(ROWS_PAD // _BLK,),
        in_specs=[
            pl.BlockSpec((_BLK, D), lambda i: (i, 0)),
            pl.BlockSpec((NC, _BLK, DH), lambda i: (0, i, 0)),
            pl.BlockSpec((1, D), lambda i: (0, 0)),
            pl.BlockSpec((_BLK, D), lambda i: (i, 0)),
            pl.BlockSpec((D, D), lambda i: (0, 0)),
        ],
        out_specs=pl.BlockSpec((_BLK, D), lambda i: (i, 0)),
        out_shape=jax.ShapeDtypeStruct((ROWS_PAD, D), jnp.float32),
    )(g, part, bias.reshape(1, D), dinv_b, w_next)


def _last_layer_body(g_ref, p_ref, b_ref, dinv_ref, wa_ref, wb_ref, bl_ref,
                     p_out, q_out):
    agg = g_ref[...] + jnp.concatenate([p_ref[0], p_ref[1]], axis=-1)
    h = dinv_ref[...] * agg + b_ref[...]
    p_out[...] = jnp.dot(h, wa_ref[...], preferred_element_type=jnp.float32)
    q_out[...] = jnp.dot(h, wb_ref[...],
                         preferred_element_type=jnp.float32) + bl_ref[...]


def _last_layer(g, part, bias, dinv_b, wa, wb, bl1):
    return pl.pallas_call(
        _last_layer_body,
        grid=(ROWS_PAD // _BLK,),
        in_specs=[
            pl.BlockSpec((_BLK, D), lambda i: (i, 0)),
            pl.BlockSpec((NC, _BLK, DH), lambda i: (0, i, 0)),
            pl.BlockSpec((1, D), lambda i: (0, 0)),
            pl.BlockSpec((_BLK, D), lambda i: (i, 0)),
            pl.BlockSpec((D, D), lambda i: (0, 0)),
            pl.BlockSpec((D, D), lambda i: (0, 0)),
            pl.BlockSpec((1, D), lambda i: (0, 0)),
        ],
        out_specs=[
            pl.BlockSpec((_BLK, D), lambda i: (i, 0)),
            pl.BlockSpec((_BLK, D), lambda i: (i, 0)),
        ],
        out_shape=[
            jax.ShapeDtypeStruct((ROWS_PAD, D), jnp.float32),
            jax.ShapeDtypeStruct((ROWS_PAD, D), jnp.float32),
        ],
    )(g, part, bias.reshape(1, D), dinv_b, wa, wb, bl1.reshape(1, D))


def _head_body(za_ref, zb_ref, w_ref, b_ref, out_ref):
    u = jnp.maximum(za_ref[...] + zb_ref[...], 0.0)
    logits = jnp.dot(u, w_ref[...],
                     preferred_element_type=jnp.float32) + b_ref[...]
    m = jnp.max(logits, axis=-1, keepdims=True)
    lse = m + jnp.log(jnp.sum(jnp.exp(logits - m), axis=-1, keepdims=True))
    out_ref[...] = logits - lse


def _head(za, zb, wl2, bl2):
    f_pad = za.shape[0]
    return pl.pallas_call(
        _head_body,
        grid=(f_pad // _BLK,),
        in_specs=[
            pl.BlockSpec((_BLK, D), lambda i: (i, 0)),
            pl.BlockSpec((_BLK, D), lambda i: (i, 0)),
            pl.BlockSpec((D, 2), lambda i: (0, 0)),
            pl.BlockSpec((1, 2), lambda i: (0, 0)),
        ],
        out_specs=pl.BlockSpec((_BLK, 2), lambda i: (i, 0)),
        out_shape=jax.ShapeDtypeStruct((f_pad, 2), jnp.float32),
    )(za, zb, wl2, bl2.reshape(1, 2))


def _prep_idx(idx, n_workers, n_slabs, slab):
    total = n_workers * n_slabs * slab
    idx = idx.astype(jnp.int32)
    idx = jnp.pad(idx, (0, total - idx.shape[0]), constant_values=PAD_IDX)
    return idx.reshape(n_workers, n_slabs, slab)


def kernel(x, edge_index, pos_edges_train, neg_edges_train, pos_edges_test,
           neg_edges_test, W0, b0, W1, b1, W2, b2, W3, b3, Wl1, bl1, Wl2, bl2):
    dst32 = _prep_idx(edge_index[1], NW, DS, SLAB)
    src_m = _prep_idx(edge_index[0], NS, MS, SLAB)
    src2 = jnp.stack([2 * src_m, 2 * src_m + 1])
    dst_m = _prep_idx(edge_index[1], NS, MS, SLAB)
    a_idx = _prep_idx(jnp.concatenate([
        pos_edges_train[0], neg_edges_train[0],
        pos_edges_test[0], neg_edges_test[0]]), NW, PS, SLABP)
    b_idx = _prep_idx(jnp.concatenate([
        pos_edges_train[1], neg_edges_train[1],
        pos_edges_test[1], neg_edges_test[1]]), NW, PS, SLABP)

    x_pad = jnp.pad(x, ((0, ROWS_PAD - N_NODES), (0, 0)))

    deg_k, msg_k, pair_k = _sc_kernels()

    degp = deg_k(dst32)
    dinv_b, g = _first_layer(degp, x_pad, W0)

    def g2(arr):
        return arr.reshape(2 * ROWS_PAD, DH)

    part = msg_k(g2(g), src2, dst_m)
    g = _mid_layer(g, part, b0, dinv_b, W1)
    part = msg_k(g2(g), src2, dst_m)
    g = _mid_layer(g, part, b1, dinv_b, W2)
    part = msg_k(g2(g), src2, dst_m)
    g = _mid_layer(g, part, b2, dinv_b, W3)
    part = msg_k(g2(g), src2, dst_m)
    p_tab, q_tab = _last_layer(g, part, b3, dinv_b, Wl1[:D], Wl1[D:], bl1)

    za, zb = pair_k(p_tab, q_tab, a_idx, b_idx)
    z = _head(za, zb, Wl2, bl2)

    n_train = 2 * pos_edges_train.shape[1]
    n_total = n_train + 2 * pos_edges_test.shape[1]
    return (z[:n_train], z[n_train:n_total])
